# Initial kernel scaffold; baseline (speedup 1.0000x reference)
import jax
import jax.numpy as jnp
from jax.experimental import pallas as pl


def kernel(x, table):
    raise NotImplementedError("write your pallas kernel here")


# speedup vs baseline: 3.2365x; 3.2365x over previous
import functools

import jax
import jax.numpy as jnp
from jax import lax
from jax.experimental import pallas as pl
from jax.experimental.pallas import tpu as pltpu
from jax.experimental.pallas import tpu_sc as plsc

D = 128
C = 128
NBUF = 2

_info = plsc.get_sparse_core_info()
_NC, _NS = _info.num_cores, _info.num_subcores
NW = _NC * _NS


def _body(idx_hbm, table_hbm, out_hbm, idx_v, rows0, rows1, g0, g1, w0, w1):
    wid = lax.axis_index("s") * _NC + lax.axis_index("c")
    nchunk = idx_hbm.shape[1]
    base = wid * nchunk * C
    rows = (rows0, rows1)
    gsem = (g0, g1)
    wsem = (w0, w1)

    pltpu.sync_copy(idx_hbm.at[wid], idx_v)

    @pl.loop(0, nchunk, step=NBUF)
    def _(j0):
        for b in range(NBUF):
            j = j0 + b

            @pl.when(j0 >= NBUF)
            def _():
                pltpu.make_async_copy(
                    rows[b],
                    out_hbm.at[pl.ds(base + (j - NBUF) * C, C)],
                    wsem[b],
                ).wait()

            pltpu.async_copy(table_hbm.at[idx_v.at[j]], rows[b], gsem[b])

        for b in range(NBUF):
            j = j0 + b
            pltpu.make_async_copy(
                table_hbm.at[idx_v.at[j]], rows[b], gsem[b]
            ).wait()
            pltpu.async_copy(
                rows[b], out_hbm.at[pl.ds(base + j * C, C)], wsem[b]
            )

    for b in range(NBUF):
        j = nchunk - NBUF + b
        pltpu.make_async_copy(
            rows[b], out_hbm.at[pl.ds(base + j * C, C)], wsem[b]
        ).wait()


@jax.jit
def kernel(x, table):
    batch, hist = x.shape
    total = batch * hist
    nchunk = total // (NW * C)
    idx3 = x.reshape(NW, nchunk, C).astype(jnp.int32)

    mesh = plsc.VectorSubcoreMesh(core_axis_name="c", subcore_axis_name="s")
    run = pl.kernel(
        _body,
        out_type=jax.ShapeDtypeStruct((total, D), jnp.float32),
        mesh=mesh,
        scratch_types=[
            pltpu.VMEM((nchunk, C), jnp.int32),
            pltpu.VMEM((C, D), jnp.float32),
            pltpu.VMEM((C, D), jnp.float32),
            pltpu.SemaphoreType.DMA,
            pltpu.SemaphoreType.DMA,
            pltpu.SemaphoreType.DMA,
            pltpu.SemaphoreType.DMA,
        ],
    )
    out = run(idx3, table)
    return out.reshape(batch, hist, D)


# speedup vs baseline: 3.3207x; 1.0260x over previous
import functools

import jax
import jax.numpy as jnp
from jax import lax
from jax.experimental import pallas as pl
from jax.experimental.pallas import tpu as pltpu
from jax.experimental.pallas import tpu_sc as plsc

D = 128
C = 128
NBUF = 5

_info = plsc.get_sparse_core_info()
_NC, _NS = _info.num_cores, _info.num_subcores
NW = _NC * _NS


def _body(idx_hbm, table_hbm, out_hbm, idx_v, *scratch):
    wid = lax.axis_index("s") * _NC + lax.axis_index("c")
    nchunk = idx_hbm.shape[1]
    base = wid * nchunk * C
    rows = scratch[:NBUF]
    gsem = scratch[NBUF:2 * NBUF]
    wsem = scratch[2 * NBUF:]

    pltpu.sync_copy(idx_hbm.at[wid], idx_v)

    @pl.loop(0, nchunk, step=NBUF)
    def _(j0):
        for b in range(NBUF):
            j = j0 + b

            @pl.when(j0 >= NBUF)
            def _():
                pltpu.make_async_copy(
                    rows[b],
                    out_hbm.at[pl.ds(base + (j - NBUF) * C, C)],
                    wsem[b],
                ).wait()

            pltpu.async_copy(table_hbm.at[idx_v.at[j]], rows[b], gsem[b])

        for b in range(NBUF):
            j = j0 + b
            pltpu.make_async_copy(
                table_hbm.at[idx_v.at[j]], rows[b], gsem[b]
            ).wait()
            pltpu.async_copy(
                rows[b], out_hbm.at[pl.ds(base + j * C, C)], wsem[b]
            )

    for b in range(NBUF):
        j = nchunk - NBUF + b
        pltpu.make_async_copy(
            rows[b], out_hbm.at[pl.ds(base + j * C, C)], wsem[b]
        ).wait()


@jax.jit
def kernel(x, table):
    batch, hist = x.shape
    total = batch * hist
    nchunk = total // (NW * C)
    idx3 = x.reshape(NW, nchunk, C).astype(jnp.int32)

    mesh = plsc.VectorSubcoreMesh(core_axis_name="c", subcore_axis_name="s")
    run = pl.kernel(
        _body,
        out_type=jax.ShapeDtypeStruct((total, D), jnp.float32),
        mesh=mesh,
        scratch_types=(
            [pltpu.VMEM((nchunk, C), jnp.int32)]
            + [pltpu.VMEM((C, D), jnp.float32)] * NBUF
            + [pltpu.SemaphoreType.DMA] * (2 * NBUF)
        ),
    )
    out = run(idx3, table)
    return out.reshape(batch, hist, D)


# speedup vs baseline: 5.9232x; 1.7837x over previous
import jax
import jax.numpy as jnp
from jax import lax
from jax.experimental import pallas as pl
from jax.experimental.pallas import tpu as pltpu
from jax.experimental.pallas import tpu_sc as plsc

D = 128
NBUF = 8

_info = plsc.get_sparse_core_info()
_NC, _NS = _info.num_cores, _info.num_subcores
NW = _NC * _NS


def _body(x_hbm, table_hbm, out_hbm, idx_v, *scratch):
    wid = lax.axis_index("s") * _NC + lax.axis_index("c")
    rows_pw = x_hbm.shape[0] // NW
    base = wid * rows_pw
    rows = scratch[:NBUF]
    gsem = scratch[NBUF:2 * NBUF]
    wsem = scratch[2 * NBUF:]

    pltpu.sync_copy(x_hbm.at[pl.ds(base, rows_pw)], idx_v)

    @pl.loop(0, rows_pw, step=NBUF)
    def _(j0):
        for b in range(NBUF):
            j = j0 + b

            @pl.when(j0 >= NBUF)
            def _():
                pltpu.make_async_copy(
                    rows[b], out_hbm.at[base + j - NBUF], wsem[b]
                ).wait()

            pltpu.async_copy(
                table_hbm.at[idx_v.at[j]], rows[b], gsem[b]
            )

        for b in range(NBUF):
            j = j0 + b
            pltpu.make_async_copy(
                table_hbm.at[idx_v.at[j]], rows[b], gsem[b]
            ).wait()
            pltpu.async_copy(rows[b], out_hbm.at[base + j], wsem[b])

    for b in range(NBUF):
        j = rows_pw - NBUF + b
        pltpu.make_async_copy(
            rows[b], out_hbm.at[base + j], wsem[b]
        ).wait()


@jax.jit
def kernel(x, table):
    batch, hist = x.shape
    rows_pw = batch // NW

    mesh = plsc.VectorSubcoreMesh(core_axis_name="c", subcore_axis_name="s")
    run = pl.kernel(
        _body,
        out_type=jax.ShapeDtypeStruct((batch, hist, D), jnp.float32),
        mesh=mesh,
        scratch_types=(
            [pltpu.VMEM((rows_pw, hist), jnp.int32)]
            + [pltpu.VMEM((hist, D), jnp.float32)] * NBUF
            + [pltpu.SemaphoreType.DMA] * (2 * NBUF)
        ),
    )
    return run(x.astype(jnp.int32), table)
